# Initial kernel scaffold; baseline (speedup 1.0000x reference)
import jax
import jax.numpy as jnp
from jax.experimental import pallas as pl


def kernel(hidden_states, gate_weight, e_score_correction_bias, w_gate, w_up, w_down, ws_gate, ws_up, ws_down):
    raise NotImplementedError("write your pallas kernel here")


# speedup vs baseline: 1.4616x; 1.4616x over previous
import jax
import jax.numpy as jnp
from jax.experimental import pallas as pl
from jax.experimental.pallas import tpu as pltpu

_TOP_K = 2
_N_GROUP = 4
_TOPK_GROUP = 2
_ROUTED_SCALING = 2.5
_B = 128
_TB = 256


def _route(x, gate_weight, bias):
    n_experts = gate_weight.shape[0]
    scores = jax.nn.sigmoid(x @ gate_weight.T)
    scores_for_choice = scores + bias[None, :]
    grp = scores_for_choice.reshape(x.shape[0], _N_GROUP, n_experts // _N_GROUP)
    group_scores = jax.lax.top_k(grp, 2)[0].sum(axis=-1)
    _, group_idx = jax.lax.top_k(group_scores, _TOPK_GROUP)
    group_mask = jnp.sum(
        jax.nn.one_hot(group_idx, _N_GROUP, dtype=scores.dtype), axis=1)
    score_mask = jnp.repeat(group_mask, n_experts // _N_GROUP, axis=1)
    masked = jnp.where(score_mask > 0, scores_for_choice,
                       jnp.zeros_like(scores_for_choice))
    _, topk_idx = jax.lax.top_k(masked, _TOP_K)
    topk_w = jnp.take_along_axis(scores, topk_idx, axis=1)
    topk_w = topk_w / (topk_w.sum(axis=-1, keepdims=True) + 1e-20)
    topk_w = topk_w * _ROUTED_SCALING
    return topk_w, topk_idx.astype(jnp.int32)


def _ffn_block(be_ref, valid_ref, xs_ref, wg_ref, wu_ref, wd_ref, out_ref):
    b = pl.program_id(0)

    @pl.when(valid_ref[b] != 0)
    def _():
        xb = xs_ref[...]
        g = jnp.dot(xb, wg_ref[0], preferred_element_type=jnp.float32)
        u = jnp.dot(xb, wu_ref[0], preferred_element_type=jnp.float32)
        h = (g * jax.nn.sigmoid(g)) * u
        out_ref[...] = jnp.dot(h, wd_ref[0], preferred_element_type=jnp.float32)

    @pl.when(valid_ref[b] == 0)
    def _():
        out_ref[...] = jnp.zeros_like(out_ref)


def _shared_combine_block(x_ref, wg_ref, wu_ref, wd_ref, a0_ref, a1_ref,
                          w0_ref, w1_ref, out_ref):
    xb = x_ref[...]
    g = jnp.dot(xb, wg_ref[...], preferred_element_type=jnp.float32)
    u = jnp.dot(xb, wu_ref[...], preferred_element_type=jnp.float32)
    h = (g * jax.nn.sigmoid(g)) * u
    s = jnp.dot(h, wd_ref[...], preferred_element_type=jnp.float32)
    out_ref[...] = s + w0_ref[...] * a0_ref[...] + w1_ref[...] * a1_ref[...]


def kernel(hidden_states, gate_weight, e_score_correction_bias, w_gate, w_up,
           w_down, ws_gate, ws_up, ws_down):
    x = hidden_states
    n_tok, d_model = x.shape
    n_experts, _, d_ff = w_gate.shape
    d_shared = ws_gate.shape[1]

    topk_w, topk_idx = _route(x, gate_weight, e_score_correction_bias)

    n_pairs = n_tok * _TOP_K
    e_flat = topk_idx.reshape(-1)
    sort_idx = jnp.argsort(e_flat).astype(jnp.int32)
    e_sorted = e_flat[sort_idx]
    tok_sorted = (sort_idx // _TOP_K).astype(jnp.int32)

    counts = jnp.sum(
        (e_flat[:, None] == jnp.arange(n_experts, dtype=e_flat.dtype)[None, :])
        .astype(jnp.int32), axis=0)
    cum = jnp.cumsum(counts)
    seg_start = cum - counts
    padded = ((counts + _B - 1) // _B) * _B
    cum_padded = jnp.cumsum(padded)
    pad_start = cum_padded - padded

    rank = jnp.arange(n_pairs, dtype=jnp.int32) - seg_start[e_sorted]
    pos = (pad_start[e_sorted] + rank).astype(jnp.int32)

    n_rows = n_pairs + n_experts * _B
    n_blocks = n_rows // _B

    tok_of_row = jnp.zeros((n_rows,), jnp.int32).at[pos].set(tok_sorted)
    inv = jnp.zeros((n_pairs,), jnp.int32).at[sort_idx].set(pos)

    block_ids = jnp.arange(n_blocks, dtype=jnp.int32) * _B
    block_e = jnp.clip(
        jnp.searchsorted(cum_padded, block_ids, side='right'),
        0, n_experts - 1).astype(jnp.int32)
    block_valid = (block_ids < cum_padded[-1]).astype(jnp.int32)

    xs = jnp.take(x, tok_of_row, axis=0)

    grid_spec = pltpu.PrefetchScalarGridSpec(
        num_scalar_prefetch=2,
        grid=(n_blocks,),
        in_specs=[
            pl.BlockSpec((_B, d_model), lambda b, be, vl: (b, 0)),
            pl.BlockSpec((1, d_model, d_ff), lambda b, be, vl: (be[b], 0, 0)),
            pl.BlockSpec((1, d_model, d_ff), lambda b, be, vl: (be[b], 0, 0)),
            pl.BlockSpec((1, d_ff, d_model), lambda b, be, vl: (be[b], 0, 0)),
        ],
        out_specs=pl.BlockSpec((_B, d_model), lambda b, be, vl: (b, 0)),
    )
    out_rows = pl.pallas_call(
        _ffn_block,
        grid_spec=grid_spec,
        out_shape=jax.ShapeDtypeStruct((n_rows, d_model), jnp.float32),
    )(block_e, block_valid, xs, w_gate, w_up, w_down)

    a0 = jnp.take(out_rows, inv[0::_TOP_K], axis=0)
    a1 = jnp.take(out_rows, inv[1::_TOP_K], axis=0)
    w0 = topk_w[:, 0:1]
    w1 = topk_w[:, 1:2]

    y = pl.pallas_call(
        _shared_combine_block,
        grid=(n_tok // _TB,),
        in_specs=[
            pl.BlockSpec((_TB, d_model), lambda i: (i, 0)),
            pl.BlockSpec((d_model, d_shared), lambda i: (0, 0)),
            pl.BlockSpec((d_model, d_shared), lambda i: (0, 0)),
            pl.BlockSpec((d_shared, d_model), lambda i: (0, 0)),
            pl.BlockSpec((_TB, d_model), lambda i: (i, 0)),
            pl.BlockSpec((_TB, d_model), lambda i: (i, 0)),
            pl.BlockSpec((_TB, 1), lambda i: (i, 0)),
            pl.BlockSpec((_TB, 1), lambda i: (i, 0)),
        ],
        out_specs=pl.BlockSpec((_TB, d_model), lambda i: (i, 0)),
        out_shape=jax.ShapeDtypeStruct((n_tok, d_model), jnp.float32),
    )(x, ws_gate, ws_up, ws_down, a0, a1, w0, w1)

    return y


# speedup vs baseline: 1.6984x; 1.1621x over previous
import jax
import jax.numpy as jnp
from jax.experimental import pallas as pl
from jax.experimental.pallas import tpu as pltpu

_TOP_K = 2
_N_GROUP = 4
_TOPK_GROUP = 2
_ROUTED_SCALING = 2.5
_B = 128
_TB = 256


def _route(x, gate_weight, bias):
    n_experts = gate_weight.shape[0]
    scores = jax.nn.sigmoid(x @ gate_weight.T)
    scores_for_choice = scores + bias[None, :]
    grp = scores_for_choice.reshape(x.shape[0], _N_GROUP, n_experts // _N_GROUP)
    group_scores = jax.lax.top_k(grp, 2)[0].sum(axis=-1)
    _, group_idx = jax.lax.top_k(group_scores, _TOPK_GROUP)
    group_mask = jnp.sum(
        jax.nn.one_hot(group_idx, _N_GROUP, dtype=scores.dtype), axis=1)
    score_mask = jnp.repeat(group_mask, n_experts // _N_GROUP, axis=1)
    masked = jnp.where(score_mask > 0, scores_for_choice,
                       jnp.zeros_like(scores_for_choice))
    _, topk_idx = jax.lax.top_k(masked, _TOP_K)
    topk_w = jnp.take_along_axis(scores, topk_idx, axis=1)
    topk_w = topk_w / (topk_w.sum(axis=-1, keepdims=True) + 1e-20)
    topk_w = topk_w * _ROUTED_SCALING
    return topk_w, topk_idx.astype(jnp.int32)


def _ffn_block(be_ref, valid_ref, xs_ref, wg_ref, wu_ref, wd_ref, out_ref):
    b = pl.program_id(0)

    @pl.when(valid_ref[b] != 0)
    def _():
        xb = xs_ref[...]
        g = jnp.dot(xb, wg_ref[0], preferred_element_type=jnp.float32)
        u = jnp.dot(xb, wu_ref[0], preferred_element_type=jnp.float32)
        h = (g * jax.nn.sigmoid(g)) * u
        out_ref[...] = jnp.dot(h, wd_ref[0], preferred_element_type=jnp.float32)

    @pl.when(valid_ref[b] == 0)
    def _():
        out_ref[...] = jnp.zeros_like(out_ref)


def _shared_combine_block(x_ref, wg_ref, wu_ref, wd_ref, a0_ref, a1_ref,
                          w0_ref, w1_ref, out_ref):
    xb = x_ref[...]
    g = jnp.dot(xb, wg_ref[...], preferred_element_type=jnp.float32)
    u = jnp.dot(xb, wu_ref[...], preferred_element_type=jnp.float32)
    h = (g * jax.nn.sigmoid(g)) * u
    s = jnp.dot(h, wd_ref[...], preferred_element_type=jnp.float32)
    out_ref[...] = s + w0_ref[...] * a0_ref[...] + w1_ref[...] * a1_ref[...]


def kernel(hidden_states, gate_weight, e_score_correction_bias, w_gate, w_up,
           w_down, ws_gate, ws_up, ws_down):
    x = hidden_states
    n_tok, d_model = x.shape
    n_experts, _, d_ff = w_gate.shape
    d_shared = ws_gate.shape[1]

    topk_w, topk_idx = _route(x, gate_weight, e_score_correction_bias)

    n_pairs = n_tok * _TOP_K
    e_flat = topk_idx.reshape(-1)
    one_hot_e = (e_flat[:, None]
                 == jnp.arange(n_experts, dtype=e_flat.dtype)[None, :])
    csum = jnp.cumsum(one_hot_e.astype(jnp.int32), axis=0)
    rank = jnp.take_along_axis(csum, e_flat[:, None].astype(jnp.int32),
                               axis=1)[:, 0] - 1
    counts = csum[-1]
    padded = ((counts + _B - 1) // _B) * _B
    cum_padded = jnp.cumsum(padded)
    pad_start = cum_padded - padded

    pos = (pad_start[e_flat] + rank).astype(jnp.int32)

    n_rows = n_pairs + n_experts * _B
    n_blocks = n_rows // _B

    tok_of_row = jnp.zeros((n_rows,), jnp.int32).at[pos].set(
        (jnp.arange(n_pairs, dtype=jnp.int32) // _TOP_K))
    inv = pos

    block_ids = jnp.arange(n_blocks, dtype=jnp.int32) * _B
    block_e = jnp.clip(
        jnp.searchsorted(cum_padded, block_ids, side='right'),
        0, n_experts - 1).astype(jnp.int32)
    block_valid = (block_ids < cum_padded[-1]).astype(jnp.int32)

    xs = jnp.take(x, tok_of_row, axis=0)

    grid_spec = pltpu.PrefetchScalarGridSpec(
        num_scalar_prefetch=2,
        grid=(n_blocks,),
        in_specs=[
            pl.BlockSpec((_B, d_model), lambda b, be, vl: (b, 0)),
            pl.BlockSpec((1, d_model, d_ff), lambda b, be, vl: (be[b], 0, 0)),
            pl.BlockSpec((1, d_model, d_ff), lambda b, be, vl: (be[b], 0, 0)),
            pl.BlockSpec((1, d_ff, d_model), lambda b, be, vl: (be[b], 0, 0)),
        ],
        out_specs=pl.BlockSpec((_B, d_model), lambda b, be, vl: (b, 0)),
    )
    out_rows = pl.pallas_call(
        _ffn_block,
        grid_spec=grid_spec,
        out_shape=jax.ShapeDtypeStruct((n_rows, d_model), jnp.float32),
    )(block_e, block_valid, xs, w_gate, w_up, w_down)

    a0 = jnp.take(out_rows, inv[0::_TOP_K], axis=0)
    a1 = jnp.take(out_rows, inv[1::_TOP_K], axis=0)
    w0 = topk_w[:, 0:1]
    w1 = topk_w[:, 1:2]

    y = pl.pallas_call(
        _shared_combine_block,
        grid=(n_tok // _TB,),
        in_specs=[
            pl.BlockSpec((_TB, d_model), lambda i: (i, 0)),
            pl.BlockSpec((d_model, d_shared), lambda i: (0, 0)),
            pl.BlockSpec((d_model, d_shared), lambda i: (0, 0)),
            pl.BlockSpec((d_shared, d_model), lambda i: (0, 0)),
            pl.BlockSpec((_TB, d_model), lambda i: (i, 0)),
            pl.BlockSpec((_TB, d_model), lambda i: (i, 0)),
            pl.BlockSpec((_TB, 1), lambda i: (i, 0)),
            pl.BlockSpec((_TB, 1), lambda i: (i, 0)),
        ],
        out_specs=pl.BlockSpec((_TB, d_model), lambda i: (i, 0)),
        out_shape=jax.ShapeDtypeStruct((n_tok, d_model), jnp.float32),
    )(x, ws_gate, ws_up, ws_down, a0, a1, w0, w1)

    return y
